# baseline (speedup 1.0000x reference)
import functools
import math

import jax
import jax.numpy as jnp
from jax.experimental import pallas as pl
from jax.experimental.pallas import tpu as pltpu

_DELTA = 0.1
_SIGMA = 0.1
_LOG2E = math.log2(math.e)
_BETA = 0.5 * _LOG2E / (_DELTA * _DELTA)
_GAMMA = _LOG2E / _SIGMA


def _dot_bf16(a, b):
    return jax.lax.dot_general(
        a.astype(jnp.bfloat16), b.astype(jnp.bfloat16),
        dimension_numbers=(((1,), (0,)), ((), ())),
        preferred_element_type=jnp.float32)


def _nms_tile_kernel(as_ref, at_ref, bs_ref, bt_ref,
                     srow_ref, scol_ref, out_ref, *, nc):
    j = pl.program_id(1)
    us = _dot_bf16(as_ref[...], bs_ref[...])
    ut = _dot_bf16(at_ref[...], bt_ref[...])
    e = _BETA * (jnp.minimum(us, 0.0) + jnp.minimum(ut, 0.0))
    o = jnp.exp2(e)
    mask = srow_ref[...] > scol_ref[...]
    p = jnp.where(mask, o, 0.0)
    psum = jnp.sum(p, axis=1, keepdims=True)
    acc = jnp.where(j == 0, 0.0, out_ref[...]) + psum
    out_ref[...] = jnp.where(j == nc - 1,
                             scol_ref[...] * jnp.exp2(-_GAMMA * acc),
                             acc)


def _round_bf16(v):
    return jax.lax.optimization_barrier(
        v.astype(jnp.bfloat16)).astype(jnp.float32)


def _split3(v):
    h = _round_bf16(v)
    r1 = v - h
    m = _round_bf16(r1)
    l = _round_bf16(r1 - m)
    return h, m, l


def _augment(x, n, n_pad):
    sq = jnp.sum(x * x, axis=1, keepdims=True)
    h, m, l = _split3(sq)
    ones = jnp.ones((n, 1), jnp.float32)
    zeros = jnp.zeros((n, 7), jnp.float32)
    a = jnp.concatenate([x, -h, -m, -l, ones, ones, ones, zeros], axis=1)
    b = jnp.concatenate([2.0 * x, ones, ones, ones, -h, -m, -l, zeros],
                        axis=1)
    pad = ((0, n_pad - n), (0, 0))
    return jnp.pad(a, pad), jnp.pad(b, pad).T


def kernel(src_points, tgt_points, scores):
    n = scores.shape[0]
    R, C = 256, 5120
    tile = max(R, C)
    n_pad = ((n + tile - 1) // tile) * tile

    xs = src_points.astype(jnp.float32)
    xt = tgt_points.astype(jnp.float32)
    a_s, b_s = _augment(xs, n, n_pad)
    a_t, b_t = _augment(xt, n, n_pad)
    s_pad = jnp.pad(scores.astype(jnp.float32), (0, n_pad - n),
                    constant_values=-jnp.inf)
    srow = s_pad[None, :]
    scol = s_pad[:, None]

    row = lambda i, j: (i, 0)
    col = lambda i, j: (0, j)

    def call_pallas(a_s, a_t, b_s, b_t, srow, scol):
        nrows = a_s.shape[0]
        grid = (nrows // R, b_s.shape[1] // C)
        return pl.pallas_call(
            functools.partial(_nms_tile_kernel, nc=grid[1]),
            grid=grid,
            in_specs=[
                pl.BlockSpec((R, 16), row),
                pl.BlockSpec((R, 16), row),
                pl.BlockSpec((16, C), col),
                pl.BlockSpec((16, C), col),
                pl.BlockSpec((1, C), col),
                pl.BlockSpec((R, 1), row),
            ],
            out_specs=pl.BlockSpec((R, 1), row),
            out_shape=jax.ShapeDtypeStruct((nrows, 1), jnp.float32),
            compiler_params=pltpu.CompilerParams(
                dimension_semantics=("parallel", "arbitrary")),
        )(a_s, a_t, b_s, b_t, srow, scol)

    out = call_pallas(a_s, a_t, b_s, b_t, srow, scol)
    return out[:n, 0]


# speedup vs baseline: 1.0550x; 1.0550x over previous
import functools
import math

import jax
import jax.numpy as jnp
from jax.experimental import pallas as pl
from jax.experimental.pallas import tpu as pltpu

_DELTA = 0.1
_SIGMA = 0.1
_LOG2E = math.log2(math.e)
_BETA = 0.5 * _LOG2E / (_DELTA * _DELTA)
_GAMMA = _LOG2E / _SIGMA


def _dot_bf16(a, b):
    return jax.lax.dot_general(
        a.astype(jnp.bfloat16), b.astype(jnp.bfloat16),
        dimension_numbers=(((1,), (0,)), ((), ())),
        preferred_element_type=jnp.float32)


def _nms_tile_kernel(as_ref, at_ref, bs_ref, bt_ref,
                     srow_ref, scol_ref, out_ref, *, nc):
    j = pl.program_id(1)
    us = _dot_bf16(as_ref[...], bs_ref[...])
    ut = _dot_bf16(at_ref[...], bt_ref[...])
    e = _BETA * (jnp.minimum(us, 0.0) + jnp.minimum(ut, 0.0))
    o = jnp.exp2(e)
    mask = srow_ref[...] > scol_ref[...]
    p = jnp.where(mask, o, 0.0)
    psum = jnp.sum(p, axis=1, keepdims=True)
    acc = jnp.where(j == 0, 0.0, out_ref[...]) + psum
    out_ref[...] = jnp.where(j == nc - 1,
                             scol_ref[...] * jnp.exp2(-_GAMMA * acc),
                             acc)


def _round_bf16(v):
    return jax.lax.optimization_barrier(
        v.astype(jnp.bfloat16)).astype(jnp.float32)


def _split3(v):
    h = _round_bf16(v)
    r1 = v - h
    m = _round_bf16(r1)
    l = _round_bf16(r1 - m)
    return h, m, l


def _augment(x, n, n_pad):
    sq = jnp.sum(x * x, axis=1, keepdims=True)
    h, m, l = _split3(sq)
    ones = jnp.ones((n, 1), jnp.float32)
    zeros = jnp.zeros((n, 7), jnp.float32)
    a = jnp.concatenate([x, -h, -m, -l, ones, ones, ones, zeros], axis=1)
    b = jnp.concatenate([2.0 * x, ones, ones, ones, -h, -m, -l, zeros],
                        axis=1)
    pad = ((0, n_pad - n), (0, 0))
    return jnp.pad(a, pad), jnp.pad(b, pad).T


def kernel(src_points, tgt_points, scores):
    n = scores.shape[0]
    R, C = 1024, 2560
    tile = max(R, C)
    n_pad = ((n + tile - 1) // tile) * tile

    xs = src_points.astype(jnp.float32)
    xt = tgt_points.astype(jnp.float32)
    a_s, b_s = _augment(xs, n, n_pad)
    a_t, b_t = _augment(xt, n, n_pad)
    s_pad = jnp.pad(scores.astype(jnp.float32), (0, n_pad - n),
                    constant_values=-jnp.inf)
    srow = s_pad[None, :]
    scol = s_pad[:, None]

    row = lambda i, j: (i, 0)
    col = lambda i, j: (0, j)

    def call_pallas(a_s, a_t, b_s, b_t, srow, scol):
        nrows = a_s.shape[0]
        grid = (nrows // R, b_s.shape[1] // C)
        return pl.pallas_call(
            functools.partial(_nms_tile_kernel, nc=grid[1]),
            grid=grid,
            in_specs=[
                pl.BlockSpec((R, 16), row),
                pl.BlockSpec((R, 16), row),
                pl.BlockSpec((16, C), col),
                pl.BlockSpec((16, C), col),
                pl.BlockSpec((1, C), col),
                pl.BlockSpec((R, 1), row),
            ],
            out_specs=pl.BlockSpec((R, 1), row),
            out_shape=jax.ShapeDtypeStruct((nrows, 1), jnp.float32),
            compiler_params=pltpu.CompilerParams(
                dimension_semantics=("parallel", "arbitrary")),
        )(a_s, a_t, b_s, b_t, srow, scol)

    out = call_pallas(a_s, a_t, b_s, b_t, srow, scol)
    return out[:n, 0]
